# baseline (speedup 1.0000x reference)
import jax
import jax.numpy as jnp
from jax.experimental import pallas as pl
from jax.experimental.pallas import tpu as pltpu

E = 16
TOP_K = 8
D = 2048
F = 1408
B = 32
NF = 2
C = F // NF
CH = C // 2
DC = D // NF
DH = DC // 2


def _ffn_kernel(rl_ref, xt_ref, w1a_ref, w1b_ref, w1ua_ref, w1ub_ref,
                w2a_ref, w2b_ref, out_ref, xt_v, combt, acct, mixa, mixb):
    e = pl.program_id(0)
    f = pl.program_id(1)

    @pl.when((e == 0) & (f == 0))
    def _init():
        xt_v[:, :] = xt_ref[:, :]
        acct[:, :] = jnp.zeros((D, B), jnp.float32)
        logits = rl_ref[:, :]
        vals = logits
        sel = jnp.zeros((B, E), jnp.float32)
        idx = jax.lax.broadcasted_iota(jnp.int32, (B, E), 1)
        for _ in range(TOP_K):
            am = jnp.argmax(vals, axis=1)
            first = idx == am[:, None]
            sel = jnp.where(first, 1.0, sel)
            vals = jnp.where(first, -jnp.inf, vals)
        mx = jnp.max(logits, axis=1, keepdims=True)
        ew = jnp.exp(logits - mx) * sel
        w = ew / jnp.sum(ew, axis=1, keepdims=True)
        combt[:, :] = w.T

    @pl.when(e < E)
    def _gate_up():
        xtb = xt_v[:, :]
        cw = combt[pl.ds(e, 1), :]
        dn = (((1,), (0,)), ((), ()))
        for half, (w1r, w1ur) in enumerate(((w1a_ref, w1ua_ref),
                                            (w1b_ref, w1ub_ref))):
            gt = jax.lax.dot_general(w1r[0], xtb, dn,
                                     preferred_element_type=jnp.float32)
            ut = jax.lax.dot_general(w1ur[0], xtb, dn,
                                     preferred_element_type=jnp.float32)
            mt = gt * jax.lax.logistic(gt) * ut * cw
            off = f * C + half * CH

            @pl.when(e % 2 == 0)
            def _():
                mixa[pl.ds(off, CH), :] = mt

            @pl.when(e % 2 == 1)
            def _():
                mixb[pl.ds(off, CH), :] = mt

    @pl.when(e > 0)
    def _down():
        dn = (((1,), (0,)), ((), ()))
        for half, w2r in enumerate((w2a_ref, w2b_ref)):
            off = f * DC + half * DH

            @pl.when(e % 2 == 1)
            def _():
                acct[pl.ds(off, DH), :] += jax.lax.dot_general(
                    w2r[0], mixa[:, :], dn,
                    preferred_element_type=jnp.float32)

            @pl.when(e % 2 == 0)
            def _():
                acct[pl.ds(off, DH), :] += jax.lax.dot_general(
                    w2r[0], mixb[:, :], dn,
                    preferred_element_type=jnp.float32)

    @pl.when((e == E) & (f == NF - 1))
    def _fin():
        out_ref[:, :] = acct[:, :]


def _w1_spec(half):
    def imap(e, f):
        fb = jnp.where(e < E, 2 * f + half, 2 + half)
        return (jnp.minimum(e, E - 1), fb, 0)
    return pl.BlockSpec((1, CH, D), imap)


def _w2_spec(half):
    def imap(e, f):
        db = jnp.where(e == 0, half, 2 * f + half)
        return (jnp.maximum(e - 1, 0), db, 0)
    return pl.BlockSpec((1, DH, F), imap)


def kernel(x, router_logits, w1, w1_up, w2):
    if x.ndim == 2:
        x = x[:, None, :]
    curr = x[:, -1, :]
    outt = pl.pallas_call(
        _ffn_kernel,
        grid=(E + 1, NF),
        in_specs=[
            pl.BlockSpec((B, E), lambda e, f: (0, 0)),
            pl.BlockSpec((D, B), lambda e, f: (0, 0)),
            _w1_spec(0), _w1_spec(1),
            _w1_spec(0), _w1_spec(1),
            _w2_spec(0), _w2_spec(1),
        ],
        out_specs=pl.BlockSpec((D, B), lambda e, f: (0, 0)),
        out_shape=jax.ShapeDtypeStruct((D, B), jnp.float32),
        scratch_shapes=[
            pltpu.VMEM((D, B), jnp.float32),
            pltpu.VMEM((E, B), jnp.float32),
            pltpu.VMEM((D, B), jnp.float32),
            pltpu.VMEM((F, B), jnp.float32),
            pltpu.VMEM((F, B), jnp.float32),
        ],
        compiler_params=pltpu.CompilerParams(
            dimension_semantics=("arbitrary", "arbitrary")),
    )(router_logits, curr.T, w1, w1, w1_up, w1_up, w2, w2)
    return outt.T.reshape(x.shape[0], 1, D)
